# baseline (speedup 1.0000x reference)
import functools

import jax
import jax.numpy as jnp
from jax import lax
from jax.experimental import pallas as pl
from jax.experimental.pallas import tpu as pltpu
from jax.experimental.pallas import tpu_sc as plsc

N = 50000
H = 64
HH = 32
E = 800000
NC, NS, L = 2, 16, 16
BLK = 128
NBLK = 392
MB = 56
NMAC = NBLK // MB
NBUF = 4
NGRP = NBLK // NBUF
E_PAD = NS * NBLK * BLK
N_ACC = 51200
STRIPE = N_ACC // NS
DUMP = N_ACC - 1

_mesh = plsc.VectorSubcoreMesh(
    core_axis_name="c", subcore_axis_name="s", num_cores=NC, num_subcores=NS)
_sc_params = pltpu.CompilerParams(use_tc_tiling_on_sc=False)

_f32 = jnp.float32


def _seg_sum4(specs):
  @functools.partial(
      pl.kernel,
      out_type=tuple(jax.ShapeDtypeStruct((N_ACC, HH), _f32)
                     for _ in range(8)),
      mesh=_mesh,
      compiler_params=_sc_params,
      scratch_types=[
          pltpu.MemorySpace.VMEM_SHARED((N_ACC, HH), _f32),
          pltpu.VMEM((NBUF, BLK), jnp.int32),
          pltpu.VMEM((NBUF, BLK), jnp.int32),
          pltpu.VMEM((NBUF, BLK), jnp.int32),
          [pltpu.VMEM((BLK, HH), _f32) for _ in range(NBUF)],
          pltpu.VMEM((BLK, HH), _f32),
          pltpu.SemaphoreType.DMA,
          [pltpu.SemaphoreType.DMA for _ in range(NBUF)],
      ],
  )
  def k(*refs):
    ins = refs[:16]
    outs = refs[16:24]
    acc, slotA, slotB, dstv, rows, zbuf, gsem, wsem = refs[24:]
    c = lax.axis_index("c")
    s = lax.axis_index("s")
    zero16 = jnp.zeros((L,), _f32)

    def zrow(i, carry):
      zbuf[i, pl.ds(0, L)] = zero16
      zbuf[i, pl.ds(L, L)] = zero16
      return carry
    lax.fori_loop(0, BLK, zrow, 0)

    def edge_loop(tab, src_h, dst_h):
      def half(src_slot, g_wait, fire):
        pltpu.sync_copy(dst_h.at[s, pl.ds(g_wait * NBUF, NBUF)], dstv)
        scs = []
        for b in range(NBUF):
          pltpu.make_async_copy(tab.at[src_slot.at[b]], rows[b], gsem).wait()
          scs.append(
              pltpu.async_copy(rows[b], acc.at[dstv.at[b]], wsem[b], add=True))
        for b in range(NBUF):
          scs[b].wait()
          if fire:
            pltpu.async_copy(tab.at[src_slot.at[b]], rows[b], gsem)

      pltpu.sync_copy(src_h.at[s, pl.ds(0, NBUF)], slotA)
      for b in range(NBUF):
        pltpu.async_copy(tab.at[slotA.at[b]], rows[b], gsem)

      def pair(p, carry):
        g0 = 2 * p
        pltpu.sync_copy(src_h.at[s, pl.ds((g0 + 1) * NBUF, NBUF)], slotB)
        half(slotB, g0, True)
        pltpu.sync_copy(src_h.at[s, pl.ds((g0 + 2) * NBUF, NBUF)], slotA)
        half(slotA, g0 + 1, True)
        return carry
      lax.fori_loop(0, (NGRP - 2) // 2, pair, 0)

      pltpu.sync_copy(src_h.at[s, pl.ds((NGRP - 1) * NBUF, NBUF)], slotB)
      half(slotB, NGRP - 2, True)
      half(slotB, NGRP - 1, False)

    for r in range(4):
      t0, t1, src_h, dst_h = ins[4 * r:4 * r + 4]
      out0, out1 = outs[2 * r], outs[2 * r + 1]

      def zcp(i, carry):
        pltpu.sync_copy(zbuf, acc.at[pl.ds(s * STRIPE + i * BLK, BLK)])
        return carry
      lax.fori_loop(0, STRIPE // BLK, zcp, 0)
      plsc.subcore_barrier()

      @pl.when(c == 0)
      def _():
        edge_loop(t0, src_h, dst_h)

      @pl.when(c == 1)
      def _():
        edge_loop(t1, src_h, dst_h)

      plsc.subcore_barrier()

      @pl.when(c == 0)
      def _():
        pltpu.sync_copy(acc.at[pl.ds(s * STRIPE, STRIPE)],
                        out0.at[pl.ds(s * STRIPE, STRIPE)])

      @pl.when(c == 1)
      def _():
        pltpu.sync_copy(acc.at[pl.ds(s * STRIPE, STRIPE)],
                        out1.at[pl.ds(s * STRIPE, STRIPE)])

  flat = [x for spec in specs for x in spec]
  res = k(*flat)
  return [(res[2 * r], res[2 * r + 1]) for r in range(4)]


def _degrees(d_ub, d_bu, d_um, d_mu):
  @functools.partial(
      pl.kernel,
      out_type=tuple(jax.ShapeDtypeStruct((N_ACC, L), _f32) for _ in range(4)),
      mesh=_mesh,
      compiler_params=_sc_params,
      scratch_types=[
          pltpu.MemorySpace.VMEM_SHARED((N_ACC, L), _f32),
          pltpu.MemorySpace.VMEM_SHARED((N_ACC, L), _f32),
          pltpu.VMEM((MB, BLK), jnp.int32),
          pltpu.VMEM((BLK, L), _f32),
          pltpu.VMEM((BLK, L), _f32),
      ],
  )
  def k(h0, h1, h2, h3, o0, o1, o2, o3, accA, accB, dstv, ones, zbuf):
    c = lax.axis_index("c")
    s = lax.axis_index("s")
    one16 = jnp.full((L,), 1.0, _f32)
    zero16 = jnp.zeros((L,), _f32)

    def fill(i, carry):
      ones[i, pl.ds(0, L)] = one16
      zbuf[i, pl.ds(0, L)] = zero16
      return carry
    lax.fori_loop(0, BLK, fill, 0)

    def zcp(i, carry):
      pltpu.sync_copy(zbuf, accA.at[pl.ds(s * STRIPE + i * BLK, BLK)])
      pltpu.sync_copy(zbuf, accB.at[pl.ds(s * STRIPE + i * BLK, BLK)])
      return carry
    lax.fori_loop(0, STRIPE // BLK, zcp, 0)
    plsc.subcore_barrier()

    def cnt_loop(dh, acc):
      def macro(m, carry):
        pltpu.sync_copy(dh.at[s, pl.ds(m * MB, MB)], dstv)

        def blk(j, c2):
          pltpu.sync_copy(ones, acc.at[dstv.at[j]], add=True)
          return c2
        lax.fori_loop(0, MB, blk, 0)
        return carry
      lax.fori_loop(0, NMAC, macro, 0)

    @pl.when(c == 0)
    def _():
      cnt_loop(h0, accA)
      cnt_loop(h1, accB)

    @pl.when(c == 1)
    def _():
      cnt_loop(h2, accA)
      cnt_loop(h3, accB)

    plsc.subcore_barrier()
    sl = pl.ds(s * STRIPE, STRIPE)

    @pl.when(c == 0)
    def _():
      pltpu.sync_copy(accA.at[sl], o0.at[sl])
      pltpu.sync_copy(accB.at[sl], o1.at[sl])

    @pl.when(c == 1)
    def _():
      pltpu.sync_copy(accA.at[sl], o2.at[sl])
      pltpu.sync_copy(accB.at[sl], o3.at[sl])

  return k(d_ub, d_bu, d_um, d_mu)


R = 400
GRID = N // R


def _mm_t(a, w):
  return lax.dot_general(a, w, (((1,), (1,)), ((), ())),
                         preferred_element_type=_f32)


def _mean(s0, s1, dg):
  s = jnp.concatenate([s0[...], s1[...]], axis=1)
  return s / jnp.maximum(dg[...][:, :1], 1.0)


def _k_l1_one(s0, s1, dg, x, wl, bl, wr, o0, o1):
  h = _mm_t(_mean(s0, s1, dg), wl[...]) + bl[...] + _mm_t(x[...], wr[...])
  h = jnp.maximum(h, 0.0)
  o0[...] = h[:, :HH]
  o1[...] = h[:, HH:]


def _k_l1_user(sa0, sa1, dga, sb0, sb1, dgb, x, wla, bla, wlb, blb, wr,
               o0, o1):
  h = (_mm_t(_mean(sa0, sa1, dga), wla[...]) + bla[...]
       + _mm_t(_mean(sb0, sb1, dgb), wlb[...]) + blb[...]
       + _mm_t(x[...], wr[...]))
  h = jnp.maximum(h, 0.0)
  o0[...] = h[:, :HH]
  o1[...] = h[:, HH:]


def _k_l2_one(s0, s1, dg, x0, x1, wl, bl, wr, lw, lb, o):
  x = jnp.concatenate([x0[...], x1[...]], axis=1)
  h = _mm_t(_mean(s0, s1, dg), wl[...]) + bl[...] + _mm_t(x, wr[...])
  o[...] = jnp.maximum(_mm_t(h, lw[...]) + lb[...], 0.0)


def _k_l2_user(sa0, sa1, dga, sb0, sb1, dgb, x0, x1, wla, bla, wlb, blb, wr,
               lw, lb, o):
  x = jnp.concatenate([x0[...], x1[...]], axis=1)
  h = (_mm_t(_mean(sa0, sa1, dga), wla[...]) + bla[...]
       + _mm_t(_mean(sb0, sb1, dgb), wlb[...]) + blb[...]
       + _mm_t(x, wr[...]))
  o[...] = jnp.maximum(_mm_t(h, lw[...]) + lb[...], 0.0)


_bs_s = pl.BlockSpec((R, HH), lambda i: (i, 0))
_bs_d = pl.BlockSpec((R, L), lambda i: (i, 0))
_bs_x = pl.BlockSpec((R, H), lambda i: (i, 0))
_bs_w = pl.BlockSpec((H, H), lambda i: (0, 0))
_bs_b = pl.BlockSpec((1, H), lambda i: (0, 0))


def _l1_one(s0, s1, dg, x, wl, bl, wr):
  return pl.pallas_call(
      _k_l1_one, grid=(GRID,),
      in_specs=[_bs_s, _bs_s, _bs_d, _bs_x, _bs_w, _bs_b, _bs_w],
      out_specs=[_bs_s, _bs_s],
      out_shape=(jax.ShapeDtypeStruct((N, HH), _f32),) * 2,
  )(s0, s1, dg, x, wl, bl, wr)


def _l1_user(sa0, sa1, dga, sb0, sb1, dgb, x, wla, bla, wlb, blb, wr):
  return pl.pallas_call(
      _k_l1_user, grid=(GRID,),
      in_specs=[_bs_s, _bs_s, _bs_d, _bs_s, _bs_s, _bs_d, _bs_x,
                _bs_w, _bs_b, _bs_w, _bs_b, _bs_w],
      out_specs=[_bs_s, _bs_s],
      out_shape=(jax.ShapeDtypeStruct((N, HH), _f32),) * 2,
  )(sa0, sa1, dga, sb0, sb1, dgb, x, wla, bla, wlb, blb, wr)


def _l2_one(s0, s1, dg, x0, x1, wl, bl, wr, lw, lb):
  return pl.pallas_call(
      _k_l2_one, grid=(GRID,),
      in_specs=[_bs_s, _bs_s, _bs_d, _bs_s, _bs_s,
                _bs_w, _bs_b, _bs_w, _bs_w, _bs_b],
      out_specs=_bs_x,
      out_shape=jax.ShapeDtypeStruct((N, H), _f32),
  )(s0, s1, dg, x0, x1, wl, bl, wr, lw, lb)


def _l2_user(sa0, sa1, dga, sb0, sb1, dgb, x0, x1, wla, bla, wlb, blb, wr,
             lw, lb):
  return pl.pallas_call(
      _k_l2_user, grid=(GRID,),
      in_specs=[_bs_s, _bs_s, _bs_d, _bs_s, _bs_s, _bs_d, _bs_s, _bs_s,
                _bs_w, _bs_b, _bs_w, _bs_b, _bs_w, _bs_w, _bs_b],
      out_specs=_bs_x,
      out_shape=jax.ShapeDtypeStruct((N, H), _f32),
  )(sa0, sa1, dga, sb0, sb1, dgb, x0, x1, wla, bla, wlb, blb, wr, lw, lb)


def kernel(params, edge_index_rb, edge_index_rm):
  p = params
  e_rb = edge_index_rb.astype(jnp.int32)
  e_rm = edge_index_rm.astype(jnp.int32)

  def pad_idx(a, fill):
    pad = jnp.full((E_PAD - E,), fill, jnp.int32)
    return jnp.concatenate([a, pad]).reshape(NS, NBLK, BLK)

  src_ub, dst_ub = pad_idx(e_rb[0], 0), pad_idx(e_rb[1], DUMP)
  src_bu, dst_bu = pad_idx(e_rb[1], 0), pad_idx(e_rb[0], DUMP)
  src_um, dst_um = pad_idx(e_rm[0], 0), pad_idx(e_rm[1], DUMP)
  src_mu, dst_mu = pad_idx(e_rm[1], 0), pad_idx(e_rm[0], DUMP)

  deg_ub, deg_bu, deg_um, deg_mu = _degrees(dst_ub, dst_bu, dst_um, dst_mu)

  xu0, xu1 = p['emb_user'][:, :HH], p['emb_user'][:, HH:]
  xb0, xb1 = p['emb_book'][:, :HH], p['emb_book'][:, HH:]
  xm0, xm1 = p['emb_movie'][:, :HH], p['emb_movie'][:, HH:]

  def b2(b):
    return b.reshape(1, H)

  ((s_ub0, s_ub1), (s_um0, s_um1), (s_bu0, s_bu1), (s_mu0, s_mu1)) = \
      _seg_sum4([(xu0, xu1, src_ub, dst_ub), (xu0, xu1, src_um, dst_um),
                 (xb0, xb1, src_bu, dst_bu), (xm0, xm1, src_mu, dst_mu)])

  b10, b11 = _l1_one(s_ub0, s_ub1, deg_ub, p['emb_book'],
                     p['c1_u_b_Wl'], b2(p['c1_u_b_bl']), p['c1_u_b_Wr'])
  m10, m11 = _l1_one(s_um0, s_um1, deg_um, p['emb_movie'],
                     p['c1_u_m_Wl'], b2(p['c1_u_m_bl']), p['c1_u_m_Wr'])
  u10, u11 = _l1_user(s_bu0, s_bu1, deg_bu, s_mu0, s_mu1, deg_mu,
                      p['emb_user'],
                      p['c1_b_u_Wl'], b2(p['c1_b_u_bl']),
                      p['c1_m_u_Wl'], b2(p['c1_m_u_bl']),
                      p['c1_b_u_Wr'] + p['c1_m_u_Wr'])

  ((t_ub0, t_ub1), (t_um0, t_um1), (t_bu0, t_bu1), (t_mu0, t_mu1)) = \
      _seg_sum4([(u10, u11, src_ub, dst_ub), (u10, u11, src_um, dst_um),
                 (b10, b11, src_bu, dst_bu), (m10, m11, src_mu, dst_mu)])

  ob = _l2_one(t_ub0, t_ub1, deg_ub, b10, b11,
               p['c2_u_b_Wl'], b2(p['c2_u_b_bl']), p['c2_u_b_Wr'],
               p['lin_book_W'], b2(p['lin_book_b']))
  om = _l2_one(t_um0, t_um1, deg_um, m10, m11,
               p['c2_u_m_Wl'], b2(p['c2_u_m_bl']), p['c2_u_m_Wr'],
               p['lin_movie_W'], b2(p['lin_movie_b']))
  ou = _l2_user(t_bu0, t_bu1, deg_bu, t_mu0, t_mu1, deg_mu, u10, u11,
                p['c2_b_u_Wl'], b2(p['c2_b_u_bl']),
                p['c2_m_u_Wl'], b2(p['c2_m_u_bl']),
                p['c2_b_u_Wr'] + p['c2_m_u_Wr'],
                p['lin_user_W'], b2(p['lin_user_b']))

  return jnp.stack([ou, ob, om])


# speedup vs baseline: 1.2637x; 1.2637x over previous
import functools

import jax
import jax.numpy as jnp
from jax import lax
from jax.experimental import pallas as pl
from jax.experimental.pallas import tpu as pltpu
from jax.experimental.pallas import tpu_sc as plsc

N = 50000
H = 64
HH = 32
E = 800000
NC, NS, L = 2, 16, 16
BLK = 128
NBLK = 392
MB = 56
NMAC = NBLK // MB
NBUF = 4
NGRP = NBLK // NBUF
E_PAD = NS * NBLK * BLK
N_ACC = 51200
STRIPE = N_ACC // NS
DUMP = N_ACC - 1

_mesh = plsc.VectorSubcoreMesh(
    core_axis_name="c", subcore_axis_name="s", num_cores=NC, num_subcores=NS)
_sc_params = pltpu.CompilerParams(use_tc_tiling_on_sc=False)

_f32 = jnp.float32


def _seg_sum(tab0, tab1, src, dst):
  @functools.partial(
      pl.kernel,
      out_type=(jax.ShapeDtypeStruct((N_ACC, HH), _f32),
                jax.ShapeDtypeStruct((N_ACC, HH), _f32)),
      mesh=_mesh,
      compiler_params=_sc_params,
      scratch_types=[
          pltpu.MemorySpace.VMEM_SHARED((N_ACC, HH), _f32),
          pltpu.VMEM((NBUF, BLK), jnp.int32),
          pltpu.VMEM((NBUF, BLK), jnp.int32),
          pltpu.VMEM((NBUF, BLK), jnp.int32),
          [pltpu.VMEM((BLK, HH), _f32) for _ in range(NBUF)],
          pltpu.VMEM((BLK, HH), _f32),
          pltpu.SemaphoreType.DMA,
          [pltpu.SemaphoreType.DMA for _ in range(NBUF)],
      ],
  )
  def k(t0, t1, src_h, dst_h, out0, out1,
        acc, slotA, slotB, dstv, rows, zbuf, gsem, wsem):
    c = lax.axis_index("c")
    s = lax.axis_index("s")
    zero16 = jnp.zeros((L,), _f32)

    def zrow(i, carry):
      zbuf[i, pl.ds(0, L)] = zero16
      zbuf[i, pl.ds(L, L)] = zero16
      return carry
    lax.fori_loop(0, BLK, zrow, 0)

    def edge_loop(tab, src_h, dst_h):
      def half(src_slot, g_wait, fire):
        pltpu.sync_copy(dst_h.at[s, pl.ds(g_wait * NBUF, NBUF)], dstv)
        scs = []
        for b in range(NBUF):
          pltpu.make_async_copy(tab.at[src_slot.at[b]], rows[b], gsem).wait()
          scs.append(
              pltpu.async_copy(rows[b], acc.at[dstv.at[b]], wsem[b], add=True))
        for b in range(NBUF):
          scs[b].wait()
          if fire:
            pltpu.async_copy(tab.at[src_slot.at[b]], rows[b], gsem)

      pltpu.sync_copy(src_h.at[s, pl.ds(0, NBUF)], slotA)
      for b in range(NBUF):
        pltpu.async_copy(tab.at[slotA.at[b]], rows[b], gsem)

      def pair(p, carry):
        g0 = 2 * p
        pltpu.sync_copy(src_h.at[s, pl.ds((g0 + 1) * NBUF, NBUF)], slotB)
        half(slotB, g0, True)
        pltpu.sync_copy(src_h.at[s, pl.ds((g0 + 2) * NBUF, NBUF)], slotA)
        half(slotA, g0 + 1, True)
        return carry
      lax.fori_loop(0, (NGRP - 2) // 2, pair, 0)

      pltpu.sync_copy(src_h.at[s, pl.ds((NGRP - 1) * NBUF, NBUF)], slotB)
      half(slotB, NGRP - 2, True)
      half(slotB, NGRP - 1, False)

    def zcp(i, carry):
      pltpu.sync_copy(zbuf, acc.at[pl.ds(s * STRIPE + i * BLK, BLK)])
      return carry
    lax.fori_loop(0, STRIPE // BLK, zcp, 0)
    plsc.subcore_barrier()

    @pl.when(c == 0)
    def _():
      edge_loop(t0, src_h, dst_h)

    @pl.when(c == 1)
    def _():
      edge_loop(t1, src_h, dst_h)

    plsc.subcore_barrier()

    @pl.when(c == 0)
    def _():
      pltpu.sync_copy(acc.at[pl.ds(s * STRIPE, STRIPE)],
                      out0.at[pl.ds(s * STRIPE, STRIPE)])

    @pl.when(c == 1)
    def _():
      pltpu.sync_copy(acc.at[pl.ds(s * STRIPE, STRIPE)],
                      out1.at[pl.ds(s * STRIPE, STRIPE)])

  return k(tab0, tab1, src, dst)


def _degrees(d_ub, d_bu, d_um, d_mu):
  @functools.partial(
      pl.kernel,
      out_type=tuple(jax.ShapeDtypeStruct((N_ACC, L), _f32) for _ in range(4)),
      mesh=_mesh,
      compiler_params=_sc_params,
      scratch_types=[
          pltpu.MemorySpace.VMEM_SHARED((N_ACC, L), _f32),
          pltpu.MemorySpace.VMEM_SHARED((N_ACC, L), _f32),
          pltpu.VMEM((MB, BLK), jnp.int32),
          pltpu.VMEM((BLK, L), _f32),
          pltpu.VMEM((BLK, L), _f32),
      ],
  )
  def k(h0, h1, h2, h3, o0, o1, o2, o3, accA, accB, dstv, ones, zbuf):
    c = lax.axis_index("c")
    s = lax.axis_index("s")
    one16 = jnp.full((L,), 1.0, _f32)
    zero16 = jnp.zeros((L,), _f32)

    def fill(i, carry):
      ones[i, pl.ds(0, L)] = one16
      zbuf[i, pl.ds(0, L)] = zero16
      return carry
    lax.fori_loop(0, BLK, fill, 0)

    def zcp(i, carry):
      pltpu.sync_copy(zbuf, accA.at[pl.ds(s * STRIPE + i * BLK, BLK)])
      pltpu.sync_copy(zbuf, accB.at[pl.ds(s * STRIPE + i * BLK, BLK)])
      return carry
    lax.fori_loop(0, STRIPE // BLK, zcp, 0)
    plsc.subcore_barrier()

    def cnt_loop(dh, acc):
      def macro(m, carry):
        pltpu.sync_copy(dh.at[s, pl.ds(m * MB, MB)], dstv)

        def blk(j, c2):
          pltpu.sync_copy(ones, acc.at[dstv.at[j]], add=True)
          return c2
        lax.fori_loop(0, MB, blk, 0)
        return carry
      lax.fori_loop(0, NMAC, macro, 0)

    @pl.when(c == 0)
    def _():
      cnt_loop(h0, accA)
      cnt_loop(h1, accB)

    @pl.when(c == 1)
    def _():
      cnt_loop(h2, accA)
      cnt_loop(h3, accB)

    plsc.subcore_barrier()
    sl = pl.ds(s * STRIPE, STRIPE)

    @pl.when(c == 0)
    def _():
      pltpu.sync_copy(accA.at[sl], o0.at[sl])
      pltpu.sync_copy(accB.at[sl], o1.at[sl])

    @pl.when(c == 1)
    def _():
      pltpu.sync_copy(accA.at[sl], o2.at[sl])
      pltpu.sync_copy(accB.at[sl], o3.at[sl])

  return k(d_ub, d_bu, d_um, d_mu)


R = 400
GRID = N // R


def _mm_t(a, w):
  return lax.dot_general(a, w, (((1,), (1,)), ((), ())),
                         preferred_element_type=_f32)


def _mean(s0, s1, dg):
  s = jnp.concatenate([s0[...], s1[...]], axis=1)
  return s / jnp.maximum(dg[...][:, :1], 1.0)


def _k_l1_one(s0, s1, dg, x, wl, bl, wr, o0, o1):
  h = _mm_t(_mean(s0, s1, dg), wl[...]) + bl[...] + _mm_t(x[...], wr[...])
  h = jnp.maximum(h, 0.0)
  o0[...] = h[:, :HH]
  o1[...] = h[:, HH:]


def _k_l1_user(sa0, sa1, dga, sb0, sb1, dgb, x, wla, bla, wlb, blb, wr,
               o0, o1):
  h = (_mm_t(_mean(sa0, sa1, dga), wla[...]) + bla[...]
       + _mm_t(_mean(sb0, sb1, dgb), wlb[...]) + blb[...]
       + _mm_t(x[...], wr[...]))
  h = jnp.maximum(h, 0.0)
  o0[...] = h[:, :HH]
  o1[...] = h[:, HH:]


def _k_l2_one(s0, s1, dg, x0, x1, wl, bl, wr, lw, lb, o):
  x = jnp.concatenate([x0[...], x1[...]], axis=1)
  h = _mm_t(_mean(s0, s1, dg), wl[...]) + bl[...] + _mm_t(x, wr[...])
  o[...] = jnp.maximum(_mm_t(h, lw[...]) + lb[...], 0.0)


def _k_l2_user(sa0, sa1, dga, sb0, sb1, dgb, x0, x1, wla, bla, wlb, blb, wr,
               lw, lb, o):
  x = jnp.concatenate([x0[...], x1[...]], axis=1)
  h = (_mm_t(_mean(sa0, sa1, dga), wla[...]) + bla[...]
       + _mm_t(_mean(sb0, sb1, dgb), wlb[...]) + blb[...]
       + _mm_t(x, wr[...]))
  o[...] = jnp.maximum(_mm_t(h, lw[...]) + lb[...], 0.0)


_bs_s = pl.BlockSpec((R, HH), lambda i: (i, 0))
_bs_d = pl.BlockSpec((R, L), lambda i: (i, 0))
_bs_x = pl.BlockSpec((R, H), lambda i: (i, 0))
_bs_w = pl.BlockSpec((H, H), lambda i: (0, 0))
_bs_b = pl.BlockSpec((1, H), lambda i: (0, 0))


def _l1_one(s0, s1, dg, x, wl, bl, wr):
  return pl.pallas_call(
      _k_l1_one, grid=(GRID,),
      in_specs=[_bs_s, _bs_s, _bs_d, _bs_x, _bs_w, _bs_b, _bs_w],
      out_specs=[_bs_s, _bs_s],
      out_shape=(jax.ShapeDtypeStruct((N, HH), _f32),) * 2,
  )(s0, s1, dg, x, wl, bl, wr)


def _l1_user(sa0, sa1, dga, sb0, sb1, dgb, x, wla, bla, wlb, blb, wr):
  return pl.pallas_call(
      _k_l1_user, grid=(GRID,),
      in_specs=[_bs_s, _bs_s, _bs_d, _bs_s, _bs_s, _bs_d, _bs_x,
                _bs_w, _bs_b, _bs_w, _bs_b, _bs_w],
      out_specs=[_bs_s, _bs_s],
      out_shape=(jax.ShapeDtypeStruct((N, HH), _f32),) * 2,
  )(sa0, sa1, dga, sb0, sb1, dgb, x, wla, bla, wlb, blb, wr)


def _l2_one(s0, s1, dg, x0, x1, wl, bl, wr, lw, lb):
  return pl.pallas_call(
      _k_l2_one, grid=(GRID,),
      in_specs=[_bs_s, _bs_s, _bs_d, _bs_s, _bs_s,
                _bs_w, _bs_b, _bs_w, _bs_w, _bs_b],
      out_specs=_bs_x,
      out_shape=jax.ShapeDtypeStruct((N, H), _f32),
  )(s0, s1, dg, x0, x1, wl, bl, wr, lw, lb)


def _l2_user(sa0, sa1, dga, sb0, sb1, dgb, x0, x1, wla, bla, wlb, blb, wr,
             lw, lb):
  return pl.pallas_call(
      _k_l2_user, grid=(GRID,),
      in_specs=[_bs_s, _bs_s, _bs_d, _bs_s, _bs_s, _bs_d, _bs_s, _bs_s,
                _bs_w, _bs_b, _bs_w, _bs_b, _bs_w, _bs_w, _bs_b],
      out_specs=_bs_x,
      out_shape=jax.ShapeDtypeStruct((N, H), _f32),
  )(sa0, sa1, dga, sb0, sb1, dgb, x0, x1, wla, bla, wlb, blb, wr, lw, lb)


def kernel(params, edge_index_rb, edge_index_rm):
  p = params
  e_rb = edge_index_rb.astype(jnp.int32)
  e_rm = edge_index_rm.astype(jnp.int32)

  def pad_idx(a, fill):
    pad = jnp.full((E_PAD - E,), fill, jnp.int32)
    return jnp.concatenate([a, pad]).reshape(NS, NBLK, BLK)

  src_ub, dst_ub = pad_idx(e_rb[0], 0), pad_idx(e_rb[1], DUMP)
  src_bu, dst_bu = pad_idx(e_rb[1], 0), pad_idx(e_rb[0], DUMP)
  src_um, dst_um = pad_idx(e_rm[0], 0), pad_idx(e_rm[1], DUMP)
  src_mu, dst_mu = pad_idx(e_rm[1], 0), pad_idx(e_rm[0], DUMP)

  deg_ub, deg_bu, deg_um, deg_mu = _degrees(dst_ub, dst_bu, dst_um, dst_mu)

  xu0, xu1 = p['emb_user'][:, :HH], p['emb_user'][:, HH:]
  xb0, xb1 = p['emb_book'][:, :HH], p['emb_book'][:, HH:]
  xm0, xm1 = p['emb_movie'][:, :HH], p['emb_movie'][:, HH:]

  def b2(b):
    return b.reshape(1, H)

  s_ub0, s_ub1 = _seg_sum(xu0, xu1, src_ub, dst_ub)
  s_um0, s_um1 = _seg_sum(xu0, xu1, src_um, dst_um)
  s_bu0, s_bu1 = _seg_sum(xb0, xb1, src_bu, dst_bu)
  s_mu0, s_mu1 = _seg_sum(xm0, xm1, src_mu, dst_mu)

  b10, b11 = _l1_one(s_ub0, s_ub1, deg_ub, p['emb_book'],
                     p['c1_u_b_Wl'], b2(p['c1_u_b_bl']), p['c1_u_b_Wr'])
  m10, m11 = _l1_one(s_um0, s_um1, deg_um, p['emb_movie'],
                     p['c1_u_m_Wl'], b2(p['c1_u_m_bl']), p['c1_u_m_Wr'])
  u10, u11 = _l1_user(s_bu0, s_bu1, deg_bu, s_mu0, s_mu1, deg_mu,
                      p['emb_user'],
                      p['c1_b_u_Wl'], b2(p['c1_b_u_bl']),
                      p['c1_m_u_Wl'], b2(p['c1_m_u_bl']),
                      p['c1_b_u_Wr'] + p['c1_m_u_Wr'])

  t_ub0, t_ub1 = _seg_sum(u10, u11, src_ub, dst_ub)
  t_um0, t_um1 = _seg_sum(u10, u11, src_um, dst_um)
  t_bu0, t_bu1 = _seg_sum(b10, b11, src_bu, dst_bu)
  t_mu0, t_mu1 = _seg_sum(m10, m11, src_mu, dst_mu)

  ob = _l2_one(t_ub0, t_ub1, deg_ub, b10, b11,
               p['c2_u_b_Wl'], b2(p['c2_u_b_bl']), p['c2_u_b_Wr'],
               p['lin_book_W'], b2(p['lin_book_b']))
  om = _l2_one(t_um0, t_um1, deg_um, m10, m11,
               p['c2_u_m_Wl'], b2(p['c2_u_m_bl']), p['c2_u_m_Wr'],
               p['lin_movie_W'], b2(p['lin_movie_b']))
  ou = _l2_user(t_bu0, t_bu1, deg_bu, t_mu0, t_mu1, deg_mu, u10, u11,
                p['c2_b_u_Wl'], b2(p['c2_b_u_bl']),
                p['c2_m_u_Wl'], b2(p['c2_m_u_bl']),
                p['c2_b_u_Wr'] + p['c2_m_u_Wr'],
                p['lin_user_W'], b2(p['lin_user_b']))

  return jnp.stack([ou, ob, om])


# speedup vs baseline: 1.2657x; 1.0015x over previous
import functools

import jax
import jax.numpy as jnp
from jax import lax
from jax.experimental import pallas as pl
from jax.experimental.pallas import tpu as pltpu
from jax.experimental.pallas import tpu_sc as plsc

N = 50000
H = 64
HH = 32
E = 800000
NC, NS, L = 2, 16, 16
BLK = 128
NBLK = 392
MB = 56
NMAC = NBLK // MB
NBUF = 4
NGRP = NBLK // NBUF
E_PAD = NS * NBLK * BLK
N_ACC = 51200
STRIPE = N_ACC // NS
DUMP = N_ACC - 1

_mesh = plsc.VectorSubcoreMesh(
    core_axis_name="c", subcore_axis_name="s", num_cores=NC, num_subcores=NS)
_sc_params = pltpu.CompilerParams(use_tc_tiling_on_sc=False)

_f32 = jnp.float32


def _seg_sum(tab0, tab1, src, dst):
  @functools.partial(
      pl.kernel,
      out_type=(jax.ShapeDtypeStruct((N_ACC, HH), _f32),
                jax.ShapeDtypeStruct((N_ACC, HH), _f32)),
      mesh=_mesh,
      compiler_params=_sc_params,
      scratch_types=[
          pltpu.MemorySpace.VMEM_SHARED((N_ACC, HH), _f32),
          pltpu.VMEM((NBUF, BLK), jnp.int32),
          pltpu.VMEM((NBUF, BLK), jnp.int32),
          pltpu.VMEM((NBUF, BLK), jnp.int32),
          [pltpu.VMEM((BLK, HH), _f32) for _ in range(NBUF)],
          pltpu.VMEM((BLK, HH), _f32),
          pltpu.SemaphoreType.DMA,
          [pltpu.SemaphoreType.DMA for _ in range(NBUF)],
      ],
  )
  def k(t0, t1, src_h, dst_h, out0, out1,
        acc, slotA, slotB, dstv, rows, zbuf, gsem, wsem):
    c = lax.axis_index("c")
    s = lax.axis_index("s")
    zero16 = jnp.zeros((L,), _f32)

    def zrow(i, carry):
      zbuf[i, pl.ds(0, L)] = zero16
      zbuf[i, pl.ds(L, L)] = zero16
      return carry
    lax.fori_loop(0, BLK, zrow, 0)

    def edge_loop(tab, src_h, dst_h):
      def half(src_slot, g_wait, fire):
        pltpu.sync_copy(dst_h.at[s, pl.ds(g_wait * NBUF, NBUF)], dstv)
        scs = []
        for b in range(NBUF):
          pltpu.make_async_copy(tab.at[src_slot.at[b]], rows[b], gsem).wait()
          scs.append(
              pltpu.async_copy(rows[b], acc.at[dstv.at[b]], wsem[b], add=True))
        for b in range(NBUF):
          scs[b].wait()
          if fire:
            pltpu.async_copy(tab.at[src_slot.at[b]], rows[b], gsem)

      pltpu.sync_copy(src_h.at[s, pl.ds(0, NBUF)], slotA)
      for b in range(NBUF):
        pltpu.async_copy(tab.at[slotA.at[b]], rows[b], gsem)

      def pair(p, carry):
        g0 = 2 * p
        pltpu.sync_copy(src_h.at[s, pl.ds((g0 + 1) * NBUF, NBUF)], slotB)
        half(slotB, g0, True)
        pltpu.sync_copy(src_h.at[s, pl.ds((g0 + 2) * NBUF, NBUF)], slotA)
        half(slotA, g0 + 1, True)
        return carry
      lax.fori_loop(0, (NGRP - 2) // 2, pair, 0)

      pltpu.sync_copy(src_h.at[s, pl.ds((NGRP - 1) * NBUF, NBUF)], slotB)
      half(slotB, NGRP - 2, True)
      half(slotB, NGRP - 1, False)

    def zcp(i, carry):
      pltpu.sync_copy(zbuf, acc.at[pl.ds(s * STRIPE + i * BLK, BLK)])
      return carry
    lax.fori_loop(0, STRIPE // BLK, zcp, 0)
    plsc.subcore_barrier()

    @pl.when(c == 0)
    def _():
      edge_loop(t0, src_h, dst_h)

    @pl.when(c == 1)
    def _():
      edge_loop(t1, src_h, dst_h)

    plsc.subcore_barrier()

    @pl.when(c == 0)
    def _():
      pltpu.sync_copy(acc.at[pl.ds(s * STRIPE, STRIPE)],
                      out0.at[pl.ds(s * STRIPE, STRIPE)])

    @pl.when(c == 1)
    def _():
      pltpu.sync_copy(acc.at[pl.ds(s * STRIPE, STRIPE)],
                      out1.at[pl.ds(s * STRIPE, STRIPE)])

  return k(tab0, tab1, src, dst)


def _degrees(d_ub, d_bu, d_um, d_mu):
  @functools.partial(
      pl.kernel,
      out_type=tuple(jax.ShapeDtypeStruct((N_ACC, L), _f32) for _ in range(4)),
      mesh=_mesh,
      compiler_params=_sc_params,
      scratch_types=[
          pltpu.MemorySpace.VMEM_SHARED((N_ACC, L), _f32),
          pltpu.MemorySpace.VMEM_SHARED((N_ACC, L), _f32),
          pltpu.VMEM((NBUF, BLK), jnp.int32),
          pltpu.VMEM((NBUF, BLK), jnp.int32),
          pltpu.VMEM((BLK, L), _f32),
          pltpu.VMEM((BLK, L), _f32),
          [pltpu.SemaphoreType.DMA for _ in range(NBUF)],
          [pltpu.SemaphoreType.DMA for _ in range(NBUF)],
      ],
  )
  def k(h0, h1, h2, h3, o0, o1, o2, o3, accA, accB, slotA, slotB, ones, zbuf,
        semA, semB):
    c = lax.axis_index("c")
    s = lax.axis_index("s")
    one16 = jnp.full((L,), 1.0, _f32)
    zero16 = jnp.zeros((L,), _f32)

    def fill(i, carry):
      ones[i, pl.ds(0, L)] = one16
      zbuf[i, pl.ds(0, L)] = zero16
      return carry
    lax.fori_loop(0, BLK, fill, 0)

    def zcp(i, carry):
      pltpu.sync_copy(zbuf, accA.at[pl.ds(s * STRIPE + i * BLK, BLK)])
      pltpu.sync_copy(zbuf, accB.at[pl.ds(s * STRIPE + i * BLK, BLK)])
      return carry
    lax.fori_loop(0, STRIPE // BLK, zcp, 0)
    plsc.subcore_barrier()

    def cnt_loop(dh, acc):
      def fire(slot, g, sems):
        return [pltpu.async_copy(ones, acc.at[slot.at[b]], sems[b], add=True)
                for b in range(NBUF)]

      pltpu.sync_copy(dh.at[s, pl.ds(0, NBUF)], slotA)
      fire(slotA, 0, semA)

      def pair(p, carry):
        g0 = 2 * p
        pltpu.sync_copy(dh.at[s, pl.ds((g0 + 1) * NBUF, NBUF)], slotB)
        dB = fire(slotB, g0 + 1, semB)
        for b in range(NBUF):
          pltpu.make_async_copy(ones, acc.at[slotA.at[b]], semA[b]).wait()
        pltpu.sync_copy(dh.at[s, pl.ds((g0 + 2) * NBUF, NBUF)], slotA)
        fire(slotA, g0 + 2, semA)
        for b in range(NBUF):
          dB[b].wait()
        return carry
      lax.fori_loop(0, (NGRP - 2) // 2, pair, 0)

      pltpu.sync_copy(dh.at[s, pl.ds((NGRP - 1) * NBUF, NBUF)], slotB)
      fire(slotB, NGRP - 1, semB)
      for b in range(NBUF):
        pltpu.make_async_copy(ones, acc.at[slotA.at[b]], semA[b]).wait()
        pltpu.make_async_copy(ones, acc.at[slotB.at[b]], semB[b]).wait()

    @pl.when(c == 0)
    def _():
      cnt_loop(h0, accA)
      cnt_loop(h1, accB)

    @pl.when(c == 1)
    def _():
      cnt_loop(h2, accA)
      cnt_loop(h3, accB)

    plsc.subcore_barrier()
    sl = pl.ds(s * STRIPE, STRIPE)

    @pl.when(c == 0)
    def _():
      pltpu.sync_copy(accA.at[sl], o0.at[sl])
      pltpu.sync_copy(accB.at[sl], o1.at[sl])

    @pl.when(c == 1)
    def _():
      pltpu.sync_copy(accA.at[sl], o2.at[sl])
      pltpu.sync_copy(accB.at[sl], o3.at[sl])

  return k(d_ub, d_bu, d_um, d_mu)


R = 400
GRID = N // R


def _mm_t(a, w):
  return lax.dot_general(a, w, (((1,), (1,)), ((), ())),
                         preferred_element_type=_f32)


def _mean(s0, s1, dg):
  s = jnp.concatenate([s0[...], s1[...]], axis=1)
  return s / jnp.maximum(dg[...][:, :1], 1.0)


def _k_l1_one(s0, s1, dg, x, wl, bl, wr, o0, o1):
  h = _mm_t(_mean(s0, s1, dg), wl[...]) + bl[...] + _mm_t(x[...], wr[...])
  h = jnp.maximum(h, 0.0)
  o0[...] = h[:, :HH]
  o1[...] = h[:, HH:]


def _k_l1_user(sa0, sa1, dga, sb0, sb1, dgb, x, wla, bla, wlb, blb, wr,
               o0, o1):
  h = (_mm_t(_mean(sa0, sa1, dga), wla[...]) + bla[...]
       + _mm_t(_mean(sb0, sb1, dgb), wlb[...]) + blb[...]
       + _mm_t(x[...], wr[...]))
  h = jnp.maximum(h, 0.0)
  o0[...] = h[:, :HH]
  o1[...] = h[:, HH:]


def _k_l2_one(s0, s1, dg, x0, x1, wl, bl, wr, lw, lb, o):
  x = jnp.concatenate([x0[...], x1[...]], axis=1)
  h = _mm_t(_mean(s0, s1, dg), wl[...]) + bl[...] + _mm_t(x, wr[...])
  o[...] = jnp.maximum(_mm_t(h, lw[...]) + lb[...], 0.0)


def _k_l2_user(sa0, sa1, dga, sb0, sb1, dgb, x0, x1, wla, bla, wlb, blb, wr,
               lw, lb, o):
  x = jnp.concatenate([x0[...], x1[...]], axis=1)
  h = (_mm_t(_mean(sa0, sa1, dga), wla[...]) + bla[...]
       + _mm_t(_mean(sb0, sb1, dgb), wlb[...]) + blb[...]
       + _mm_t(x, wr[...]))
  o[...] = jnp.maximum(_mm_t(h, lw[...]) + lb[...], 0.0)


_bs_s = pl.BlockSpec((R, HH), lambda i: (i, 0))
_bs_d = pl.BlockSpec((R, L), lambda i: (i, 0))
_bs_x = pl.BlockSpec((R, H), lambda i: (i, 0))
_bs_w = pl.BlockSpec((H, H), lambda i: (0, 0))
_bs_b = pl.BlockSpec((1, H), lambda i: (0, 0))


def _l1_one(s0, s1, dg, x, wl, bl, wr):
  return pl.pallas_call(
      _k_l1_one, grid=(GRID,),
      in_specs=[_bs_s, _bs_s, _bs_d, _bs_x, _bs_w, _bs_b, _bs_w],
      out_specs=[_bs_s, _bs_s],
      out_shape=(jax.ShapeDtypeStruct((N, HH), _f32),) * 2,
  )(s0, s1, dg, x, wl, bl, wr)


def _l1_user(sa0, sa1, dga, sb0, sb1, dgb, x, wla, bla, wlb, blb, wr):
  return pl.pallas_call(
      _k_l1_user, grid=(GRID,),
      in_specs=[_bs_s, _bs_s, _bs_d, _bs_s, _bs_s, _bs_d, _bs_x,
                _bs_w, _bs_b, _bs_w, _bs_b, _bs_w],
      out_specs=[_bs_s, _bs_s],
      out_shape=(jax.ShapeDtypeStruct((N, HH), _f32),) * 2,
  )(sa0, sa1, dga, sb0, sb1, dgb, x, wla, bla, wlb, blb, wr)


def _l2_one(s0, s1, dg, x0, x1, wl, bl, wr, lw, lb):
  return pl.pallas_call(
      _k_l2_one, grid=(GRID,),
      in_specs=[_bs_s, _bs_s, _bs_d, _bs_s, _bs_s,
                _bs_w, _bs_b, _bs_w, _bs_w, _bs_b],
      out_specs=_bs_x,
      out_shape=jax.ShapeDtypeStruct((N, H), _f32),
  )(s0, s1, dg, x0, x1, wl, bl, wr, lw, lb)


def _l2_user(sa0, sa1, dga, sb0, sb1, dgb, x0, x1, wla, bla, wlb, blb, wr,
             lw, lb):
  return pl.pallas_call(
      _k_l2_user, grid=(GRID,),
      in_specs=[_bs_s, _bs_s, _bs_d, _bs_s, _bs_s, _bs_d, _bs_s, _bs_s,
                _bs_w, _bs_b, _bs_w, _bs_b, _bs_w, _bs_w, _bs_b],
      out_specs=_bs_x,
      out_shape=jax.ShapeDtypeStruct((N, H), _f32),
  )(sa0, sa1, dga, sb0, sb1, dgb, x0, x1, wla, bla, wlb, blb, wr, lw, lb)


def kernel(params, edge_index_rb, edge_index_rm):
  p = params
  e_rb = edge_index_rb.astype(jnp.int32)
  e_rm = edge_index_rm.astype(jnp.int32)

  def pad_idx(a, fill):
    pad = jnp.full((E_PAD - E,), fill, jnp.int32)
    return jnp.concatenate([a, pad]).reshape(NS, NBLK, BLK)

  src_ub, dst_ub = pad_idx(e_rb[0], 0), pad_idx(e_rb[1], DUMP)
  src_bu, dst_bu = pad_idx(e_rb[1], 0), pad_idx(e_rb[0], DUMP)
  src_um, dst_um = pad_idx(e_rm[0], 0), pad_idx(e_rm[1], DUMP)
  src_mu, dst_mu = pad_idx(e_rm[1], 0), pad_idx(e_rm[0], DUMP)

  deg_ub, deg_bu, deg_um, deg_mu = _degrees(dst_ub, dst_bu, dst_um, dst_mu)

  xu0, xu1 = p['emb_user'][:, :HH], p['emb_user'][:, HH:]
  xb0, xb1 = p['emb_book'][:, :HH], p['emb_book'][:, HH:]
  xm0, xm1 = p['emb_movie'][:, :HH], p['emb_movie'][:, HH:]

  def b2(b):
    return b.reshape(1, H)

  s_ub0, s_ub1 = _seg_sum(xu0, xu1, src_ub, dst_ub)
  s_um0, s_um1 = _seg_sum(xu0, xu1, src_um, dst_um)
  s_bu0, s_bu1 = _seg_sum(xb0, xb1, src_bu, dst_bu)
  s_mu0, s_mu1 = _seg_sum(xm0, xm1, src_mu, dst_mu)

  b10, b11 = _l1_one(s_ub0, s_ub1, deg_ub, p['emb_book'],
                     p['c1_u_b_Wl'], b2(p['c1_u_b_bl']), p['c1_u_b_Wr'])
  m10, m11 = _l1_one(s_um0, s_um1, deg_um, p['emb_movie'],
                     p['c1_u_m_Wl'], b2(p['c1_u_m_bl']), p['c1_u_m_Wr'])
  u10, u11 = _l1_user(s_bu0, s_bu1, deg_bu, s_mu0, s_mu1, deg_mu,
                      p['emb_user'],
                      p['c1_b_u_Wl'], b2(p['c1_b_u_bl']),
                      p['c1_m_u_Wl'], b2(p['c1_m_u_bl']),
                      p['c1_b_u_Wr'] + p['c1_m_u_Wr'])

  t_ub0, t_ub1 = _seg_sum(u10, u11, src_ub, dst_ub)
  t_um0, t_um1 = _seg_sum(u10, u11, src_um, dst_um)
  t_bu0, t_bu1 = _seg_sum(b10, b11, src_bu, dst_bu)
  t_mu0, t_mu1 = _seg_sum(m10, m11, src_mu, dst_mu)

  ob = _l2_one(t_ub0, t_ub1, deg_ub, b10, b11,
               p['c2_u_b_Wl'], b2(p['c2_u_b_bl']), p['c2_u_b_Wr'],
               p['lin_book_W'], b2(p['lin_book_b']))
  om = _l2_one(t_um0, t_um1, deg_um, m10, m11,
               p['c2_u_m_Wl'], b2(p['c2_u_m_bl']), p['c2_u_m_Wr'],
               p['lin_movie_W'], b2(p['lin_movie_b']))
  ou = _l2_user(t_bu0, t_bu1, deg_bu, t_mu0, t_mu1, deg_mu, u10, u11,
                p['c2_b_u_Wl'], b2(p['c2_b_u_bl']),
                p['c2_m_u_Wl'], b2(p['c2_m_u_bl']),
                p['c2_b_u_Wr'] + p['c2_m_u_Wr'],
                p['lin_user_W'], b2(p['lin_user_b']))

  return jnp.stack([ou, ob, om])
